# baseline (speedup 1.0000x reference)
import jax
import jax.numpy as jnp
from jax import lax
from jax.experimental import pallas as pl

B, H, S, L, D = 16, 50, 100, 2, 256
K = 32
P = 5


def _dg(a, b, dims):
    return lax.dot_general(a, b, (dims, ((), ())),
                           preferred_element_type=jnp.float32,
                           precision=lax.Precision.HIGHEST)


def _body(x_ref, u_ref, out_ref, kid_ref):
    u = u_ref[0]
    usq = jnp.sum(u * u)
    nu = u / jnp.maximum(jnp.sqrt(usq), 1e-12)

    ii = lax.broadcasted_iota(jnp.int32, (S, S), 0)
    jj = lax.broadcasted_iota(jnp.int32, (S, S), 1)
    eye = (ii == jj).astype(jnp.float32)
    kk = lax.broadcasted_iota(jnp.int32, (S, K), 1).astype(jnp.float32)
    iota_c = lax.broadcasted_iota(jnp.int32, (S, 1), 0).astype(jnp.float32)

    for p in range(P):
        x = x_ref[p]
        mean = (x[:, :D] + x[:, D:]) * 0.5
        msq = jnp.sum(mean * mean, axis=1, keepdims=True)
        nm = mean / jnp.maximum(jnp.sqrt(msq), 1e-12)
        nub = jnp.broadcast_to(nu, (8, D)).astype(jnp.bfloat16)
        nmb = nm.astype(jnp.bfloat16)
        sc_c = lax.dot_general(
            nmb, nub, (((1,), (1,)), ((), ())),
            preferred_element_type=jnp.float32)[:, 0:1]
        sc_r = lax.dot_general(
            nub, nmb, (((1,), (1,)), ((), ())),
            preferred_element_type=jnp.float32)[0:1, :]

        beats = ((sc_r > sc_c) | ((sc_r == sc_c) & (jj < ii)))
        rank_c = jnp.sum(beats.astype(jnp.float32), axis=1, keepdims=True)
        selT = (rank_c == kk).astype(jnp.float32)

        kid_row = jnp.sum(selT * iota_c, axis=0, keepdims=True)
        wsel = (selT * sc_c).astype(jnp.bfloat16)
        wout = lax.dot_general(
            wsel, x.astype(jnp.bfloat16),
            (((0,), (0,)), ((), ())),
            preferred_element_type=jnp.float32)

        out_ref[p] = wout
        kid_ref[p] = kid_row.astype(jnp.int32)


def kernel(news_embedding, user_repr):
    x = news_embedding.reshape(B * H, S, L * D)
    u = user_repr
    grid = (B * H) // P

    w, kid = pl.pallas_call(
        _body,
        grid=(grid,),
        in_specs=[
            pl.BlockSpec((P, S, L * D), lambda g: (g, 0, 0)),
            pl.BlockSpec((1, 1, D), lambda g: (g // (H // P), 0, 0)),
        ],
        out_specs=[
            pl.BlockSpec((P, K, L * D), lambda g: (g, 0, 0)),
            pl.BlockSpec((P, 1, K), lambda g: (g, 0, 0)),
        ],
        out_shape=[
            jax.ShapeDtypeStruct((B * H, K, L * D), jnp.float32),
            jax.ShapeDtypeStruct((B * H, 1, K), jnp.int32),
        ],
    )(x, u)

    return (w.reshape(B, H, K, L, D), kid.reshape(B, H, K))


# speedup vs baseline: 1.0634x; 1.0634x over previous
import jax
import jax.numpy as jnp
from jax import lax
from jax.experimental import pallas as pl

B, H, S, L, D = 16, 50, 100, 2, 256
K = 32
P = 50


def _dg(a, b, dims):
    return lax.dot_general(a, b, (dims, ((), ())),
                           preferred_element_type=jnp.float32,
                           precision=lax.Precision.HIGHEST)


def _body(x_ref, u_ref, out_ref, kid_ref):
    u = u_ref[0]
    usq = jnp.sum(u * u)
    nu = u / jnp.maximum(jnp.sqrt(usq), 1e-12)

    ii = lax.broadcasted_iota(jnp.int32, (S, S), 0)
    jj = lax.broadcasted_iota(jnp.int32, (S, S), 1)
    eye = (ii == jj).astype(jnp.float32)
    kk = lax.broadcasted_iota(jnp.int32, (S, K), 1).astype(jnp.float32)
    iota_c = lax.broadcasted_iota(jnp.int32, (S, 1), 0).astype(jnp.float32)

    for p in range(P):
        x = x_ref[p]
        mean = (x[:, :D] + x[:, D:]) * 0.5
        msq = jnp.sum(mean * mean, axis=1, keepdims=True)
        nm = mean / jnp.maximum(jnp.sqrt(msq), 1e-12)
        nub = jnp.broadcast_to(nu, (8, D)).astype(jnp.bfloat16)
        nmb = nm.astype(jnp.bfloat16)
        sc_c = lax.dot_general(
            nmb, nub, (((1,), (1,)), ((), ())),
            preferred_element_type=jnp.float32)[:, 0:1]
        sc_r = lax.dot_general(
            nub, nmb, (((1,), (1,)), ((), ())),
            preferred_element_type=jnp.float32)[0:1, :]

        beats = ((sc_r > sc_c) | ((sc_r == sc_c) & (jj < ii)))
        rank_c = jnp.sum(beats.astype(jnp.float32), axis=1, keepdims=True)
        selT = (rank_c == kk).astype(jnp.float32)

        kid_row = jnp.sum(selT * iota_c, axis=0, keepdims=True)
        wsel = (selT * sc_c).astype(jnp.bfloat16)
        wout = lax.dot_general(
            wsel, x.astype(jnp.bfloat16),
            (((0,), (0,)), ((), ())),
            preferred_element_type=jnp.float32)

        out_ref[p] = wout
        kid_ref[p] = kid_row.astype(jnp.int32)


def kernel(news_embedding, user_repr):
    x = news_embedding.reshape(B * H, S, L * D)
    u = user_repr
    grid = (B * H) // P

    w, kid = pl.pallas_call(
        _body,
        grid=(grid,),
        in_specs=[
            pl.BlockSpec((P, S, L * D), lambda g: (g, 0, 0)),
            pl.BlockSpec((1, 1, D), lambda g: (g // (H // P), 0, 0)),
        ],
        out_specs=[
            pl.BlockSpec((P, K, L * D), lambda g: (g, 0, 0)),
            pl.BlockSpec((P, 1, K), lambda g: (g, 0, 0)),
        ],
        out_shape=[
            jax.ShapeDtypeStruct((B * H, K, L * D), jnp.float32),
            jax.ShapeDtypeStruct((B * H, 1, K), jnp.int32),
        ],
    )(x, u)

    return (w.reshape(B, H, K, L, D), kid.reshape(B, H, K))


# speedup vs baseline: 1.0649x; 1.0014x over previous
import jax
import jax.numpy as jnp
from jax import lax
from jax.experimental import pallas as pl
from jax.experimental.pallas import tpu as pltpu

B, H, S, L, D = 16, 50, 100, 2, 256
K = 32
P = 50


def _dg(a, b, dims):
    return lax.dot_general(a, b, (dims, ((), ())),
                           preferred_element_type=jnp.float32,
                           precision=lax.Precision.HIGHEST)


def _body(x_ref, u_ref, out_ref, kid_ref):
    u = u_ref[0]
    usq = jnp.sum(u * u)
    nu = u / jnp.maximum(jnp.sqrt(usq), 1e-12)

    ii = lax.broadcasted_iota(jnp.int32, (S, S), 0)
    jj = lax.broadcasted_iota(jnp.int32, (S, S), 1)
    eye = (ii == jj).astype(jnp.float32)
    kk = lax.broadcasted_iota(jnp.int32, (S, K), 1).astype(jnp.float32)
    iota_c = lax.broadcasted_iota(jnp.int32, (S, 1), 0).astype(jnp.float32)

    for p in range(P):
        x = x_ref[p]
        mean = (x[:, :D] + x[:, D:]) * 0.5
        msq = jnp.sum(mean * mean, axis=1, keepdims=True)
        nm = mean / jnp.maximum(jnp.sqrt(msq), 1e-12)
        nub = jnp.broadcast_to(nu, (8, D)).astype(jnp.bfloat16)
        nmb = nm.astype(jnp.bfloat16)
        sc_c = lax.dot_general(
            nmb, nub, (((1,), (1,)), ((), ())),
            preferred_element_type=jnp.float32)[:, 0:1]
        sc_r = lax.dot_general(
            nub, nmb, (((1,), (1,)), ((), ())),
            preferred_element_type=jnp.float32)[0:1, :]

        beats = ((sc_r > sc_c) | ((sc_r == sc_c) & (jj < ii)))
        rank_c = jnp.sum(beats.astype(jnp.float32), axis=1, keepdims=True)
        selT = (rank_c == kk).astype(jnp.float32)

        kid_row = jnp.sum(selT * iota_c, axis=0, keepdims=True)
        wsel = (selT * sc_c).astype(jnp.bfloat16)
        wout = lax.dot_general(
            wsel, x.astype(jnp.bfloat16),
            (((0,), (0,)), ((), ())),
            preferred_element_type=jnp.float32)

        out_ref[p] = wout
        kid_ref[p] = kid_row.astype(jnp.int32)


def kernel(news_embedding, user_repr):
    x = news_embedding.reshape(B * H, S, L * D)
    u = user_repr
    grid = (B * H) // P

    w, kid = pl.pallas_call(
        _body,
        grid=(grid,),
        in_specs=[
            pl.BlockSpec((P, S, L * D), lambda g: (g, 0, 0)),
            pl.BlockSpec((1, 1, D), lambda g: (g // (H // P), 0, 0)),
        ],
        out_specs=[
            pl.BlockSpec((P, K, L * D), lambda g: (g, 0, 0)),
            pl.BlockSpec((P, 1, K), lambda g: (g, 0, 0)),
        ],
        out_shape=[
            jax.ShapeDtypeStruct((B * H, K, L * D), jnp.float32),
            jax.ShapeDtypeStruct((B * H, 1, K), jnp.int32),
        ],
        compiler_params=pltpu.CompilerParams(
            vmem_limit_bytes=100 * 1024 * 1024),
    )(x, u)

    return (w.reshape(B, H, K, L, D), kid.reshape(B, H, K))
